# baseline (speedup 1.0000x reference)
import functools

import jax
import jax.numpy as jnp
from jax import lax
from jax.experimental import pallas as pl
from jax.experimental.pallas import tpu as pltpu
from jax.experimental.pallas import tpu_sc as plsc

BATCH = 16384
D = 64
PD = 2 * D
L = 16
NC, NS = 2, 16
NW = NC * NS
BPW = BATCH // NW
CHUNK = 128
NCHUNK = BPW // CHUNK
GPC = CHUNK // L

_mesh = plsc.VectorSubcoreMesh(core_axis_name="c", subcore_axis_name="s")


@functools.partial(
    pl.kernel,
    mesh=_mesh,
    out_type=jax.ShapeDtypeStruct((BATCH,), jnp.float32),
    compiler_params=pltpu.CompilerParams(needs_layout_passes=False),
    scratch_types=[
        pltpu.VMEM((BPW,), jnp.int32),
        pltpu.VMEM((BPW,), jnp.int32),
        pltpu.VMEM((BPW,), jnp.int32),
        pltpu.VMEM((BPW,), jnp.int32),
        pltpu.VMEM((2, CHUNK, PD), jnp.float32),
        pltpu.VMEM((2, CHUNK, PD), jnp.float32),
        pltpu.VMEM((BPW,), jnp.float32),
        pltpu.SemaphoreType.DMA,
        pltpu.SemaphoreType.DMA,
        pltpu.SemaphoreType.DMA,
        pltpu.SemaphoreType.DMA,
    ],
)
def _bi_encoder(uidx_hbm, bidx_hbm, utab_hbm, btab_hbm, out_hbm,
                uidx_v, bidx_v, uphys_v, bphys_v, ubuf, bbuf, out_v,
                sem_u0, sem_u1, sem_b0, sem_b1):
    wid = lax.axis_index("s") * NC + lax.axis_index("c")
    base = wid * BPW

    pltpu.sync_copy(uidx_hbm.at[pl.ds(base, BPW)], uidx_v)
    pltpu.sync_copy(bidx_hbm.at[pl.ds(base, BPW)], bidx_v)

    def pbody(i, carry):
        sl = pl.ds(i * L, L)
        uphys_v[sl] = lax.shift_right_logical(uidx_v[sl], 1)
        bphys_v[sl] = lax.shift_right_logical(bidx_v[sl], 1)
        return carry

    lax.fori_loop(0, BPW // L, pbody, 0)

    sems_u = (sem_u0, sem_u1)
    sems_b = (sem_b0, sem_b1)

    def fire(c):
        slot = c % 2
        sl = pl.ds(c * CHUNK, CHUNK)
        cu = pltpu.async_copy(utab_hbm.at[uphys_v.at[sl]], ubuf.at[slot],
                              sems_u[slot])
        cb = pltpu.async_copy(btab_hbm.at[bphys_v.at[sl]], bbuf.at[slot],
                              sems_b[slot])
        return cu, cb

    viota = lax.iota(jnp.int32, L)
    inflight = fire(0)
    for c in range(NCHUNK):
        cu, cb = inflight
        if c + 1 < NCHUNK:
            nxt = fire(c + 1)
        cu.wait()
        cb.wait()
        if c + 1 < NCHUNK:
            inflight = nxt
        slot = c % 2
        ub = ubuf.at[slot]
        bb = bbuf.at[slot]

        def gbody(g, carry):
            rows = g * L + viota
            isl = pl.ds(c * CHUNK + g * L, L)
            ucol = (uidx_v[isl] & 1) * D
            bcol = (bidx_v[isl] & 1) * D

            def dbody(d, acc):
                uu = plsc.load_gather(ub, [rows, ucol + d])
                vv = plsc.load_gather(bb, [rows, bcol + d])
                return acc + uu * vv

            acc = lax.fori_loop(0, D, dbody, jnp.zeros((L,), jnp.float32),
                                unroll=8)
            out_v[isl] = 1.0 / (1.0 + jnp.exp(-acc))
            return carry

        lax.fori_loop(0, GPC, gbody, 0)

    pltpu.sync_copy(out_v, out_hbm.at[pl.ds(base, BPW)])


def kernel(user_indices, book_indices, user_table, book_table):
    n_u, n_b = user_table.shape[0], book_table.shape[0]
    ut2 = user_table.reshape(n_u // 2, PD)
    bt2 = book_table.reshape(n_b // 2, PD)
    return _bi_encoder(user_indices.astype(jnp.int32),
                       book_indices.astype(jnp.int32),
                       ut2, bt2)
